# Initial kernel scaffold; baseline (speedup 1.0000x reference)
import jax
import jax.numpy as jnp
from jax.experimental import pallas as pl


def kernel(x, edge_index, edge_attr, batch, u, emb_W, emb_b, b1_W, b1_b, be_W, be_b, b2_W, b2_b, fc_W, fc_b, hb_W1, hb_b1, hb_W2, hb_b2, he_W1, he_b1, he_W2, he_b2):
    raise NotImplementedError("write your pallas kernel here")


# speedup vs baseline: 1.3821x; 1.3821x over previous
import functools

import jax
import jax.numpy as jnp
from jax import lax
from jax.experimental import pallas as pl
from jax.experimental.pallas import tpu as pltpu
from jax.experimental.pallas import tpu_sc as plsc

N = 50000
E = 800000
G = 512
NAI = 4
NAF = 64
NRBF = 10
NCONV = 3
NH = 64

NSC = 2
NTILE = 16

NP = 50176
EC = 128
EPT = 50048
NCHUNK = EPT // EC
E_PAD = EPT * NTILE

HALF = NP // 2
SPROWS = HALF + 16
ZPT = SPROWS // NTILE
ZCH = ZPT // EC
ZREM = ZPT - ZCH * EC
OCH = 14
OC = HALF // NTILE // OCH

GP = 528
NPT = NP // (NSC * NTILE)
PCH = 14
PC = NPT // PCH

_STEP = 6.0 / (NRBF - 1)
_COEFF = -0.5 / _STEP ** 2


def _ef_body(attr_ref, beW_ref, beb_ref, ef0_ref, ef1_ref, ef2_ref):
    a = attr_ref[0, 0, :]
    offs = jnp.arange(NRBF, dtype=jnp.int32).astype(jnp.float32) * _STEP
    d = a[:, None] - offs[None, :]
    ea = jnp.exp(_COEFF * d * d)
    outs = (ef0_ref, ef1_ref, ef2_ref)
    for i in range(NCONV):
        outs[i][...] = ea @ beW_ref[i] + beb_ref[i][None, :]


def _tc_ef(attr_p, be_W, be_b):
    eshape = jax.ShapeDtypeStruct((E_PAD, NAF), jnp.float32)
    attr2 = attr_p.reshape(E_PAD // 2048, 1, 2048)
    return pl.pallas_call(
        _ef_body,
        grid=(E_PAD // 2048,),
        in_specs=[
            pl.BlockSpec((1, 1, 2048), lambda i: (i, 0, 0)),
            pl.BlockSpec((NCONV, NRBF, NAF), lambda i: (0, 0, 0)),
            pl.BlockSpec((NCONV, NAF), lambda i: (0, 0)),
        ],
        out_specs=[pl.BlockSpec((2048, NAF), lambda i: (i, 0))] * NCONV,
        out_shape=[eshape] * NCONV,
    )(attr2, be_W, be_b)


def _emb_body(x_ref, eW_ref, eb_ref, w1_ref, b1_ref, h_ref, hs_ref):
    h = x_ref[...] @ eW_ref[...] + eb_ref[...]
    h_ref[...] = h
    hs_ref[...] = h @ w1_ref[...] + b1_ref[...]


def _tc_emb(xp, emb_W, emb_b, w1, b1):
    hshape = jax.ShapeDtypeStruct((NP, NAF), jnp.float32)
    return pl.pallas_call(
        _emb_body,
        grid=(NP // 1024,),
        in_specs=[
            pl.BlockSpec((1024, NAI), lambda i: (i, 0)),
            pl.BlockSpec((NAI, NAF), lambda i: (0, 0)),
            pl.BlockSpec((1, NAF), lambda i: (0, 0)),
            pl.BlockSpec((NAF, NAF), lambda i: (0, 0)),
            pl.BlockSpec((1, NAF), lambda i: (0, 0)),
        ],
        out_specs=[pl.BlockSpec((1024, NAF), lambda i: (i, 0))] * 2,
        out_shape=[hshape, hshape],
    )(xp, emb_W, emb_b.reshape(1, NAF), w1, b1.reshape(1, NAF))


def _softplus(t):
    return jnp.maximum(t, 0.0) + jnp.log1p(jnp.exp(-jnp.abs(t)))


def _post_body(h_ref, agg_ref, w2_ref, b2_ref, w1_ref, b1_ref, h_out, hs_out):
    t = h_ref[...] + agg_ref[...] @ w2_ref[...] + b2_ref[...]
    hn = _softplus(t)
    h_out[...] = hn
    hs_out[...] = hn @ w1_ref[...] + b1_ref[...]


def _tc_post(h, agg, w2, b2, w1, b1):
    hshape = jax.ShapeDtypeStruct((NP, NAF), jnp.float32)
    return pl.pallas_call(
        _post_body,
        grid=(NP // 1024,),
        in_specs=[
            pl.BlockSpec((1024, NAF), lambda i: (i, 0)),
            pl.BlockSpec((1024, NAF), lambda i: (i, 0)),
            pl.BlockSpec((NAF, NAF), lambda i: (0, 0)),
            pl.BlockSpec((1, NAF), lambda i: (0, 0)),
            pl.BlockSpec((NAF, NAF), lambda i: (0, 0)),
            pl.BlockSpec((1, NAF), lambda i: (0, 0)),
        ],
        out_specs=[pl.BlockSpec((1024, NAF), lambda i: (i, 0))] * 2,
        out_shape=[hshape, hshape],
    )(h, agg, w2, b2.reshape(1, NAF), w1, b1.reshape(1, NAF))


def _post_last_body(h_ref, agg_ref, w2_ref, b2_ref, h_out):
    t = h_ref[...] + agg_ref[...] @ w2_ref[...] + b2_ref[...]
    h_out[...] = _softplus(t)


def _tc_post_last(h, agg, w2, b2):
    return pl.pallas_call(
        _post_last_body,
        grid=(NP // 1024,),
        in_specs=[
            pl.BlockSpec((1024, NAF), lambda i: (i, 0)),
            pl.BlockSpec((1024, NAF), lambda i: (i, 0)),
            pl.BlockSpec((NAF, NAF), lambda i: (0, 0)),
            pl.BlockSpec((1, NAF), lambda i: (0, 0)),
        ],
        out_specs=pl.BlockSpec((1024, NAF), lambda i: (i, 0)),
        out_shape=jax.ShapeDtypeStruct((NP, NAF), jnp.float32),
    )(h, agg, w2, b2.reshape(1, NAF))


def _head_body(ps_ref, pc_ref, u_ref, fcW_ref, fcb_ref,
               hbW1_ref, hbb1_ref, hbW2_ref, hbb2_ref,
               heW1_ref, heb1_ref, heW2_ref, heb2_ref,
               bg_ref, eh_ref):
    sums = ps_ref[0:G, :] + ps_ref[GP:GP + G, :]
    cnt = pc_ref[0:G, 0:1] + pc_ref[GP:GP + G, 0:1]
    c = sums / jnp.maximum(cnt, 1.0)
    fcW = fcW_ref[...]
    z = c @ fcW[0:NAF, :] + u_ref[...] @ fcW[NAF:NAF + 2, :] + fcb_ref[...]
    z = jnp.maximum(z, 0.0)
    bg_raw = (jnp.maximum(z @ hbW1_ref[...] + hbb1_ref[...], 0.0)
              @ hbW2_ref[...] + hbb2_ref[...])
    bg_ref[...] = jnp.log1p(jnp.maximum(bg_raw, 0.0))
    eh_ref[...] = (jnp.maximum(z @ heW1_ref[...] + heb1_ref[...], 0.0)
                   @ heW2_ref[...] + heb2_ref[...])


def _tc_head(psum, pcnt, u, fc_W, fc_b, hb_W1, hb_b1, hb_W2, hb_b2,
             he_W1, he_b1, he_W2, he_b2):
    whole = lambda a: pl.BlockSpec(a.shape, lambda: tuple(0 for _ in a.shape))
    args = (psum, pcnt, u, fc_W, fc_b.reshape(1, 2 * NH),
            hb_W1, hb_b1.reshape(1, NH), hb_W2, hb_b2.reshape(1, 1),
            he_W1, he_b1.reshape(1, NH), he_W2, he_b2.reshape(1, 1))
    oshape = jax.ShapeDtypeStruct((G, 1), jnp.float32)
    return pl.pallas_call(
        _head_body,
        in_specs=[whole(a) for a in args],
        out_specs=[pl.BlockSpec((G, 1), lambda: (0, 0))] * 2,
        out_shape=[oshape, oshape],
    )(*args)


@functools.lru_cache(maxsize=None)
def _sc_mesh():
    return plsc.VectorSubcoreMesh(
        core_axis_name="c", subcore_axis_name="s", num_cores=NSC,
        num_subcores=NTILE)


def _conv_body(hs_hbm, ef_hbm, src_hbm, dst_hbm, agg_hbm,
               srcv, dstv, idxv, hsr, efr, msg, aggsp):
    c = lax.axis_index("c")
    s = lax.axis_index("s")
    base = c * HALF

    @pl.loop(0, EC)
    def _zfill(r):
        for j in range(NAF // 16):
            msg[r, pl.ds(j * 16, 16)] = jnp.zeros((16,), jnp.float32)

    @pl.loop(0, ZCH)
    def _zcopy(r):
        row0 = s * ZPT + r * EC
        pltpu.sync_copy(msg, aggsp.at[pl.ds(row0, EC)])

    pltpu.sync_copy(msg.at[pl.ds(0, ZREM)],
                    aggsp.at[pl.ds(s * ZPT + ZCH * EC, ZREM)])

    plsc.subcore_barrier()

    tb = s * EPT

    @pl.loop(0, NCHUNK)
    def _chunk(k):
        off = pl.multiple_of(tb + k * EC, EC)
        pltpu.sync_copy(src_hbm.at[pl.ds(off, EC)], srcv)
        pltpu.sync_copy(dst_hbm.at[pl.ds(off, EC)], dstv)
        pltpu.sync_copy(hs_hbm.at[srcv], hsr)
        pltpu.sync_copy(ef_hbm.at[pl.ds(off, EC)], efr)

        @pl.loop(0, EC // 16)
        def _idx(g):
            o = pl.multiple_of(g * 16, 16)
            d = dstv[pl.ds(o, 16)]
            loc = d - base
            inr = (loc >= 0) & (loc < HALF)
            idxv[pl.ds(o, 16)] = jnp.where(inr, loc, HALF)

        @pl.loop(0, EC, unroll=2)
        def _mul(r):
            for j in range(NAF // 16):
                sl = pl.ds(j * 16, 16)
                msg[r, sl] = hsr[r, sl] * efr[r, sl]

        pltpu.sync_copy(msg, aggsp.at[idxv], add=True)

    plsc.subcore_barrier()

    @pl.loop(0, OCH)
    def _out(r):
        row0 = pl.multiple_of((s * OCH + r) * OC, 16)
        pltpu.sync_copy(aggsp.at[pl.ds(row0, OC)],
                        agg_hbm.at[pl.ds(base + row0, OC)])


@functools.lru_cache(maxsize=None)
def _sc_conv_kernel():
    return pl.kernel(
        _conv_body,
        out_type=jax.ShapeDtypeStruct((NP, NAF), jnp.float32),
        mesh=_sc_mesh(),
        compiler_params=pltpu.CompilerParams(use_tc_tiling_on_sc=False),
        scratch_types=[
            pltpu.VMEM((EC,), jnp.int32),
            pltpu.VMEM((EC,), jnp.int32),
            pltpu.VMEM((EC,), jnp.int32),
            pltpu.VMEM((EC, NAF), jnp.float32),
            pltpu.VMEM((EC, NAF), jnp.float32),
            pltpu.VMEM((EC, NAF), jnp.float32),
            pltpu.VMEM_SHARED((SPROWS, NAF), jnp.float32),
        ],
    )


def _sc_conv(hs_hbm, ef_hbm, src_hbm, dst_hbm):
    return _sc_conv_kernel()(hs_hbm, ef_hbm, src_hbm, dst_hbm)


def _pool_body(h_hbm, batch_hbm, psum_hbm, pcnt_hbm,
               bidv, hbuf, obuf, psum_sp, pcnt_sp):
    c = lax.axis_index("c")
    s = lax.axis_index("s")

    @pl.loop(0, PC)
    def _fill(r):
        for j in range(NAF // 16):
            hbuf[r, pl.ds(j * 16, 16)] = jnp.zeros((16,), jnp.float32)
            obuf[r, pl.ds(j * 16, 16)] = jnp.ones((16,), jnp.float32)

    zrows = GP // NTILE
    pltpu.sync_copy(hbuf.at[pl.ds(0, zrows)], psum_sp.at[pl.ds(s * zrows, zrows)])
    pltpu.sync_copy(hbuf.at[pl.ds(0, zrows)], pcnt_sp.at[pl.ds(s * zrows, zrows)])
    plsc.subcore_barrier()

    nb0 = (c * NTILE + s) * NPT

    @pl.loop(0, PCH)
    def _chunk(k):
        off = pl.multiple_of(nb0 + k * PC, 16)
        pltpu.sync_copy(batch_hbm.at[pl.ds(off, PC)], bidv)
        pltpu.sync_copy(h_hbm.at[pl.ds(off, PC)], hbuf)
        pltpu.sync_copy(hbuf, psum_sp.at[bidv], add=True)
        pltpu.sync_copy(obuf, pcnt_sp.at[bidv], add=True)

    plsc.subcore_barrier()

    row0 = s * zrows
    pltpu.sync_copy(psum_sp.at[pl.ds(row0, zrows)],
                    psum_hbm.at[pl.ds(c * GP + row0, zrows)])
    pltpu.sync_copy(pcnt_sp.at[pl.ds(row0, zrows)],
                    pcnt_hbm.at[pl.ds(c * GP + row0, zrows)])


@functools.lru_cache(maxsize=None)
def _sc_pool_kernel():
    return pl.kernel(
        _pool_body,
        out_type=[jax.ShapeDtypeStruct((NSC * GP, NAF), jnp.float32),
                  jax.ShapeDtypeStruct((NSC * GP, NAF), jnp.float32)],
        mesh=_sc_mesh(),
        compiler_params=pltpu.CompilerParams(use_tc_tiling_on_sc=False),
        scratch_types=[
            pltpu.VMEM((PC,), jnp.int32),
            pltpu.VMEM((PC, NAF), jnp.float32),
            pltpu.VMEM((PC, NAF), jnp.float32),
            pltpu.VMEM_SHARED((GP, NAF), jnp.float32),
            pltpu.VMEM_SHARED((GP, NAF), jnp.float32),
        ],
    )


def _sc_pool(h_hbm, batch_hbm):
    return _sc_pool_kernel()(h_hbm, batch_hbm)


def kernel(x, edge_index, edge_attr, batch, u,
           emb_W, emb_b, b1_W, b1_b, be_W, be_b, b2_W, b2_b,
           fc_W, fc_b, hb_W1, hb_b1, hb_W2, hb_b2, he_W1, he_b1, he_W2, he_b2):
    xp = jnp.pad(x, ((0, NP - N), (0, 0)))
    srcp = jnp.pad(edge_index[0], (0, E_PAD - E))
    dstp = jnp.pad(edge_index[1], (0, E_PAD - E), constant_values=3 * HALF)
    attrp = jnp.pad(edge_attr, (0, E_PAD - E))
    batchp = jnp.pad(batch, (0, NP - N), constant_values=G)

    efs = _tc_ef(attrp, be_W, be_b)
    h, hs = _tc_emb(xp, emb_W, emb_b, b1_W[0], b1_b[0])
    for i in range(NCONV):
        agg = _sc_conv(hs, efs[i], srcp, dstp)
        if i + 1 < NCONV:
            h, hs = _tc_post(h, agg, b2_W[i], b2_b[i], b1_W[i + 1], b1_b[i + 1])
        else:
            h = _tc_post_last(h, agg, b2_W[i], b2_b[i])

    psum, pcnt = _sc_pool(h, batchp)
    return _tc_head(psum, pcnt, u, fc_W, fc_b, hb_W1, hb_b1, hb_W2, hb_b2,
                    he_W1, he_b1, he_W2, he_b2)


# speedup vs baseline: 1.7674x; 1.2788x over previous
import functools

import jax
import jax.numpy as jnp
from jax import lax
from jax.experimental import pallas as pl
from jax.experimental.pallas import tpu as pltpu
from jax.experimental.pallas import tpu_sc as plsc

N = 50000
E = 800000
G = 512
NAI = 4
NAF = 64
NRBF = 10
NCONV = 3
NH = 64

NSC = 2
NTILE = 16

NP = 50176
EC = 64
EPT = 50048
NCHUNK = EPT // EC
E_PAD = EPT * NTILE

HALF = NP // 2
SPROWS = HALF + 16
ZPT = SPROWS // NTILE
ZCH = ZPT // EC
ZREM = ZPT - ZCH * EC
OCH = 14
OC = HALF // NTILE // OCH

GP = 528
NPT = NP // (NSC * NTILE)
PCH = 14
PC = NPT // PCH

_STEP = 6.0 / (NRBF - 1)
_COEFF = -0.5 / _STEP ** 2


def _ef_body(attr_ref, beW_ref, beb_ref, ef0_ref, ef1_ref, ef2_ref):
    a = attr_ref[0, 0, :]
    offs = jnp.arange(NRBF, dtype=jnp.int32).astype(jnp.float32) * _STEP
    d = a[:, None] - offs[None, :]
    ea = jnp.exp(_COEFF * d * d)
    outs = (ef0_ref, ef1_ref, ef2_ref)
    for i in range(NCONV):
        outs[i][...] = ea @ beW_ref[i] + beb_ref[i][None, :]


def _tc_ef(attr_p, be_W, be_b):
    eshape = jax.ShapeDtypeStruct((E_PAD, NAF), jnp.float32)
    attr2 = attr_p.reshape(E_PAD // 2048, 1, 2048)
    return pl.pallas_call(
        _ef_body,
        grid=(E_PAD // 2048,),
        in_specs=[
            pl.BlockSpec((1, 1, 2048), lambda i: (i, 0, 0)),
            pl.BlockSpec((NCONV, NRBF, NAF), lambda i: (0, 0, 0)),
            pl.BlockSpec((NCONV, NAF), lambda i: (0, 0)),
        ],
        out_specs=[pl.BlockSpec((2048, NAF), lambda i: (i, 0))] * NCONV,
        out_shape=[eshape] * NCONV,
    )(attr2, be_W, be_b)


def _emb_body(x_ref, eW_ref, eb_ref, w1_ref, b1_ref, h_ref, hs_ref):
    h = x_ref[...] @ eW_ref[...] + eb_ref[...]
    h_ref[...] = h
    hs_ref[...] = h @ w1_ref[...] + b1_ref[...]


def _tc_emb(xp, emb_W, emb_b, w1, b1):
    hshape = jax.ShapeDtypeStruct((NP, NAF), jnp.float32)
    return pl.pallas_call(
        _emb_body,
        grid=(NP // 1024,),
        in_specs=[
            pl.BlockSpec((1024, NAI), lambda i: (i, 0)),
            pl.BlockSpec((NAI, NAF), lambda i: (0, 0)),
            pl.BlockSpec((1, NAF), lambda i: (0, 0)),
            pl.BlockSpec((NAF, NAF), lambda i: (0, 0)),
            pl.BlockSpec((1, NAF), lambda i: (0, 0)),
        ],
        out_specs=[pl.BlockSpec((1024, NAF), lambda i: (i, 0))] * 2,
        out_shape=[hshape, hshape],
    )(xp, emb_W, emb_b.reshape(1, NAF), w1, b1.reshape(1, NAF))


def _softplus(t):
    return jnp.maximum(t, 0.0) + jnp.log1p(jnp.exp(-jnp.abs(t)))


def _post_body(h_ref, agg_ref, w2_ref, b2_ref, w1_ref, b1_ref, h_out, hs_out):
    t = h_ref[...] + agg_ref[...] @ w2_ref[...] + b2_ref[...]
    hn = _softplus(t)
    h_out[...] = hn
    hs_out[...] = hn @ w1_ref[...] + b1_ref[...]


def _tc_post(h, agg, w2, b2, w1, b1):
    hshape = jax.ShapeDtypeStruct((NP, NAF), jnp.float32)
    return pl.pallas_call(
        _post_body,
        grid=(NP // 1024,),
        in_specs=[
            pl.BlockSpec((1024, NAF), lambda i: (i, 0)),
            pl.BlockSpec((1024, NAF), lambda i: (i, 0)),
            pl.BlockSpec((NAF, NAF), lambda i: (0, 0)),
            pl.BlockSpec((1, NAF), lambda i: (0, 0)),
            pl.BlockSpec((NAF, NAF), lambda i: (0, 0)),
            pl.BlockSpec((1, NAF), lambda i: (0, 0)),
        ],
        out_specs=[pl.BlockSpec((1024, NAF), lambda i: (i, 0))] * 2,
        out_shape=[hshape, hshape],
    )(h, agg, w2, b2.reshape(1, NAF), w1, b1.reshape(1, NAF))


def _post_last_body(h_ref, agg_ref, w2_ref, b2_ref, h_out):
    t = h_ref[...] + agg_ref[...] @ w2_ref[...] + b2_ref[...]
    h_out[...] = _softplus(t)


def _tc_post_last(h, agg, w2, b2):
    return pl.pallas_call(
        _post_last_body,
        grid=(NP // 1024,),
        in_specs=[
            pl.BlockSpec((1024, NAF), lambda i: (i, 0)),
            pl.BlockSpec((1024, NAF), lambda i: (i, 0)),
            pl.BlockSpec((NAF, NAF), lambda i: (0, 0)),
            pl.BlockSpec((1, NAF), lambda i: (0, 0)),
        ],
        out_specs=pl.BlockSpec((1024, NAF), lambda i: (i, 0)),
        out_shape=jax.ShapeDtypeStruct((NP, NAF), jnp.float32),
    )(h, agg, w2, b2.reshape(1, NAF))


def _head_body(ps_ref, pc_ref, u_ref, fcW_ref, fcb_ref,
               hbW1_ref, hbb1_ref, hbW2_ref, hbb2_ref,
               heW1_ref, heb1_ref, heW2_ref, heb2_ref,
               bg_ref, eh_ref):
    sums = ps_ref[0:G, :] + ps_ref[GP:GP + G, :]
    cnt = pc_ref[0:G, 0:1] + pc_ref[GP:GP + G, 0:1]
    c = sums / jnp.maximum(cnt, 1.0)
    fcW = fcW_ref[...]
    z = c @ fcW[0:NAF, :] + u_ref[...] @ fcW[NAF:NAF + 2, :] + fcb_ref[...]
    z = jnp.maximum(z, 0.0)
    bg_raw = (jnp.maximum(z @ hbW1_ref[...] + hbb1_ref[...], 0.0)
              @ hbW2_ref[...] + hbb2_ref[...])
    bg_ref[...] = jnp.log1p(jnp.maximum(bg_raw, 0.0))
    eh_ref[...] = (jnp.maximum(z @ heW1_ref[...] + heb1_ref[...], 0.0)
                   @ heW2_ref[...] + heb2_ref[...])


def _tc_head(psum, pcnt, u, fc_W, fc_b, hb_W1, hb_b1, hb_W2, hb_b2,
             he_W1, he_b1, he_W2, he_b2):
    whole = lambda a: pl.BlockSpec(a.shape, lambda: tuple(0 for _ in a.shape))
    args = (psum, pcnt, u, fc_W, fc_b.reshape(1, 2 * NH),
            hb_W1, hb_b1.reshape(1, NH), hb_W2, hb_b2.reshape(1, 1),
            he_W1, he_b1.reshape(1, NH), he_W2, he_b2.reshape(1, 1))
    oshape = jax.ShapeDtypeStruct((G, 1), jnp.float32)
    return pl.pallas_call(
        _head_body,
        in_specs=[whole(a) for a in args],
        out_specs=[pl.BlockSpec((G, 1), lambda: (0, 0))] * 2,
        out_shape=[oshape, oshape],
    )(*args)


@functools.lru_cache(maxsize=None)
def _sc_mesh():
    return plsc.VectorSubcoreMesh(
        core_axis_name="c", subcore_axis_name="s", num_cores=NSC,
        num_subcores=NTILE)


def _conv_body(hs_hbm, ef_hbm, src_hbm, dst_hbm, agg_hbm,
               srcv0, dstv0, idxv0, hsr0, efr0, msg0,
               srcv1, dstv1, idxv1, hsr1, efr1, msg1,
               lsem0, gsem0, ssem0, lsem1, gsem1, ssem1, aggsp):
    c = lax.axis_index("c")
    s = lax.axis_index("s")
    base = c * HALF
    B = ((srcv0, dstv0, idxv0, hsr0, efr0, msg0, lsem0, gsem0, ssem0),
         (srcv1, dstv1, idxv1, hsr1, efr1, msg1, lsem1, gsem1, ssem1))

    @pl.loop(0, EC)
    def _zfill(r):
        for j in range(NAF // 16):
            msg0[r, pl.ds(j * 16, 16)] = jnp.zeros((16,), jnp.float32)

    @pl.loop(0, ZCH)
    def _zcopy(r):
        row0 = s * ZPT + r * EC
        pltpu.sync_copy(msg0, aggsp.at[pl.ds(row0, EC)])

    pltpu.sync_copy(msg0.at[pl.ds(0, ZREM)],
                    aggsp.at[pl.ds(s * ZPT + ZCH * EC, ZREM)])

    plsc.subcore_barrier()

    tb = s * EPT

    def lin_issue(k, buf):
        srcv, dstv, _, _, efr, _, lsem, _, _ = buf
        off = pl.multiple_of(tb + k * EC, EC)
        pltpu.async_copy(src_hbm.at[pl.ds(off, EC)], srcv, lsem)
        pltpu.async_copy(dst_hbm.at[pl.ds(off, EC)], dstv, lsem)
        pltpu.async_copy(ef_hbm.at[pl.ds(off, EC)], efr, lsem)

    def lin_wait(k, buf):
        srcv, dstv, _, _, efr, _, lsem, _, _ = buf
        off = pl.multiple_of(tb + k * EC, EC)
        pltpu.make_async_copy(src_hbm.at[pl.ds(off, EC)], srcv, lsem).wait()
        pltpu.make_async_copy(dst_hbm.at[pl.ds(off, EC)], dstv, lsem).wait()
        pltpu.make_async_copy(ef_hbm.at[pl.ds(off, EC)], efr, lsem).wait()

    def step(k, buf, nxt, do_scatter_wait, issue_pred=True):
        srcv, dstv, idxv, hsr, efr, msg, lsem, gsem, ssem = buf
        lin_wait(k, buf)
        pltpu.async_copy(hs_hbm.at[srcv], hsr, gsem)
        if nxt is not None:
            if issue_pred is True:
                lin_issue(k + 1, nxt)
            else:
                pl.when(issue_pred)(lambda: lin_issue(k + 1, nxt))
        if do_scatter_wait is not None:
            w = lambda: pltpu.make_async_copy(
                msg, aggsp.at[idxv], ssem).wait()
            if do_scatter_wait is True:
                w()
            else:
                pl.when(do_scatter_wait)(w)

        @pl.loop(0, EC // 16)
        def _idx(g):
            o = pl.multiple_of(g * 16, 16)
            d = dstv[pl.ds(o, 16)]
            loc = d - base
            inr = (loc >= 0) & (loc < HALF)
            idxv[pl.ds(o, 16)] = jnp.where(inr, loc, HALF)

        pltpu.make_async_copy(hs_hbm.at[srcv], hsr, gsem).wait()

        @pl.loop(0, EC, unroll=2)
        def _mul(r):
            for j in range(NAF // 16):
                sl = pl.ds(j * 16, 16)
                msg[r, sl] = hsr[r, sl] * efr[r, sl]

        pltpu.async_copy(msg, aggsp.at[idxv], ssem, add=True)

    lin_issue(0, B[0])

    @pl.loop(0, NCHUNK // 2)
    def _round(r):
        step(2 * r, B[0], B[1], r >= 1)
        step(2 * r + 1, B[1], B[0], r >= 1,
             issue_pred=r < NCHUNK // 2 - 1)

    pltpu.make_async_copy(msg0, aggsp.at[idxv0], ssem0).wait()
    pltpu.make_async_copy(msg1, aggsp.at[idxv1], ssem1).wait()

    plsc.subcore_barrier()

    @pl.loop(0, OCH)
    def _out(r):
        row0 = pl.multiple_of((s * OCH + r) * OC, 16)
        pltpu.sync_copy(aggsp.at[pl.ds(row0, OC)],
                        agg_hbm.at[pl.ds(base + row0, OC)])


@functools.lru_cache(maxsize=None)
def _sc_conv_kernel():
    return pl.kernel(
        _conv_body,
        out_type=jax.ShapeDtypeStruct((NP, NAF), jnp.float32),
        mesh=_sc_mesh(),
        compiler_params=pltpu.CompilerParams(use_tc_tiling_on_sc=False),
        scratch_types=(
            [pltpu.VMEM((EC,), jnp.int32)] * 3
            + [pltpu.VMEM((EC, NAF), jnp.float32)] * 3
            + [pltpu.VMEM((EC,), jnp.int32)] * 3
            + [pltpu.VMEM((EC, NAF), jnp.float32)] * 3
            + [pltpu.SemaphoreType.DMA] * 6
            + [pltpu.VMEM_SHARED((SPROWS, NAF), jnp.float32)]
        ),
    )


def _sc_conv(hs_hbm, ef_hbm, src_hbm, dst_hbm):
    return _sc_conv_kernel()(hs_hbm, ef_hbm, src_hbm, dst_hbm)


def _pool_body(h_hbm, batch_hbm, psum_hbm, pcnt_hbm,
               bidv, hbuf, obuf, psum_sp, pcnt_sp):
    c = lax.axis_index("c")
    s = lax.axis_index("s")

    @pl.loop(0, PC)
    def _fill(r):
        for j in range(NAF // 16):
            hbuf[r, pl.ds(j * 16, 16)] = jnp.zeros((16,), jnp.float32)
            obuf[r, pl.ds(j * 16, 16)] = jnp.ones((16,), jnp.float32)

    zrows = GP // NTILE
    pltpu.sync_copy(hbuf.at[pl.ds(0, zrows)], psum_sp.at[pl.ds(s * zrows, zrows)])
    pltpu.sync_copy(hbuf.at[pl.ds(0, zrows)], pcnt_sp.at[pl.ds(s * zrows, zrows)])
    plsc.subcore_barrier()

    nb0 = (c * NTILE + s) * NPT

    @pl.loop(0, PCH)
    def _chunk(k):
        off = pl.multiple_of(nb0 + k * PC, 16)
        pltpu.sync_copy(batch_hbm.at[pl.ds(off, PC)], bidv)
        pltpu.sync_copy(h_hbm.at[pl.ds(off, PC)], hbuf)
        pltpu.sync_copy(hbuf, psum_sp.at[bidv], add=True)
        pltpu.sync_copy(obuf, pcnt_sp.at[bidv], add=True)

    plsc.subcore_barrier()

    row0 = s * zrows
    pltpu.sync_copy(psum_sp.at[pl.ds(row0, zrows)],
                    psum_hbm.at[pl.ds(c * GP + row0, zrows)])
    pltpu.sync_copy(pcnt_sp.at[pl.ds(row0, zrows)],
                    pcnt_hbm.at[pl.ds(c * GP + row0, zrows)])


@functools.lru_cache(maxsize=None)
def _sc_pool_kernel():
    return pl.kernel(
        _pool_body,
        out_type=[jax.ShapeDtypeStruct((NSC * GP, NAF), jnp.float32),
                  jax.ShapeDtypeStruct((NSC * GP, NAF), jnp.float32)],
        mesh=_sc_mesh(),
        compiler_params=pltpu.CompilerParams(use_tc_tiling_on_sc=False),
        scratch_types=[
            pltpu.VMEM((PC,), jnp.int32),
            pltpu.VMEM((PC, NAF), jnp.float32),
            pltpu.VMEM((PC, NAF), jnp.float32),
            pltpu.VMEM_SHARED((GP, NAF), jnp.float32),
            pltpu.VMEM_SHARED((GP, NAF), jnp.float32),
        ],
    )


def _sc_pool(h_hbm, batch_hbm):
    return _sc_pool_kernel()(h_hbm, batch_hbm)


def kernel(x, edge_index, edge_attr, batch, u,
           emb_W, emb_b, b1_W, b1_b, be_W, be_b, b2_W, b2_b,
           fc_W, fc_b, hb_W1, hb_b1, hb_W2, hb_b2, he_W1, he_b1, he_W2, he_b2):
    xp = jnp.pad(x, ((0, NP - N), (0, 0)))
    srcp = jnp.pad(edge_index[0], (0, E_PAD - E))
    dstp = jnp.pad(edge_index[1], (0, E_PAD - E), constant_values=3 * HALF)
    attrp = jnp.pad(edge_attr, (0, E_PAD - E))
    batchp = jnp.pad(batch, (0, NP - N), constant_values=G)

    efs = _tc_ef(attrp, be_W, be_b)
    h, hs = _tc_emb(xp, emb_W, emb_b, b1_W[0], b1_b[0])
    for i in range(NCONV):
        agg = _sc_conv(hs, efs[i], srcp, dstp)
        if i + 1 < NCONV:
            h, hs = _tc_post(h, agg, b2_W[i], b2_b[i], b1_W[i + 1], b1_b[i + 1])
        else:
            h = _tc_post_last(h, agg, b2_W[i], b2_b[i])

    psum, pcnt = _sc_pool(h, batchp)
    return _tc_head(psum, pcnt, u, fc_W, fc_b, hb_W1, hb_b1, hb_W2, hb_b2,
                    he_W1, he_b1, he_W2, he_b2)
